# Initial kernel scaffold; baseline (speedup 1.0000x reference)
import functools

import jax
import jax.numpy as jnp
from jax import lax
from jax.experimental import pallas as pl
from jax.experimental.pallas import tpu as pltpu
from jax.experimental.pallas import tpu_sc as plsc

N = 10000
D = 128
E = 320000

NC, NS = 2, 16
NW = NC * NS
C = 128
EPT_CHUNKS = 79
EPT = EPT_CHUNKS * C
E_PAD = NW * EPT
N_PAD = 10240
RPT = N_PAD // NS
DEG_W = 16

_mesh = plsc.VectorSubcoreMesh(core_axis_name="c", subcore_axis_name="s")


@functools.partial(
    pl.kernel,
    out_type=jax.ShapeDtypeStruct((NC, N_PAD, DEG_W), jnp.float32),
    mesh=_mesh,
    scratch_types=[
        pltpu.VMEM((EPT_CHUNKS, C), jnp.int32),
        pltpu.VMEM((C, DEG_W), jnp.float32),
        pltpu.VMEM((RPT, DEG_W), jnp.float32),
    ],
)
def _deg_kernel(dst_hbm, ones_hbm, zeros_hbm, deg_hbm, dst_v, ones_v, buf_v):
    cid = lax.axis_index("c")
    sid = lax.axis_index("s")
    tid = cid * NS + sid
    pltpu.sync_copy(dst_hbm.at[tid], dst_v)
    pltpu.sync_copy(ones_hbm, ones_v)
    pltpu.sync_copy(zeros_hbm, buf_v)

    def scope(acc):
        base = sid * RPT
        pltpu.sync_copy(buf_v, acc.at[pl.ds(base, RPT)])
        plsc.subcore_barrier()

        def body(j, carry):
            pltpu.sync_copy(ones_v, acc.at[dst_v.at[j]], add=True)
            return carry

        lax.fori_loop(0, EPT_CHUNKS, body, 0)
        plsc.subcore_barrier()
        pltpu.sync_copy(acc.at[pl.ds(base, RPT)], buf_v)
        pltpu.sync_copy(buf_v, deg_hbm.at[cid].at[pl.ds(base, RPT)])

    pl.run_scoped(scope, pltpu.VMEM_SHARED((N_PAD, DEG_W), jnp.float32))


@functools.partial(
    pl.kernel,
    out_type=jax.ShapeDtypeStruct((NC, N_PAD, D), jnp.float32),
    mesh=_mesh,
    scratch_types=[
        pltpu.VMEM((EPT_CHUNKS, C), jnp.int32),
        pltpu.VMEM((EPT_CHUNKS, C), jnp.int32),
        pltpu.VMEM((C, D), jnp.float32),
        pltpu.SemaphoreType.DMA,
    ],
)
def _scatter_kernel(h2_hbm, src_hbm, dst_hbm, zeros_hbm, acc_hbm,
                    src_v, dst_v, buf_v, sem):
    cid = lax.axis_index("c")
    sid = lax.axis_index("s")
    tid = cid * NS + sid
    pltpu.sync_copy(src_hbm.at[tid], src_v)
    pltpu.sync_copy(dst_hbm.at[tid], dst_v)

    def scope(acc):
        pltpu.sync_copy(zeros_hbm, buf_v)
        for k in range(RPT // C):
            pltpu.sync_copy(buf_v, acc.at[pl.ds(sid * RPT + k * C, C)])
        plsc.subcore_barrier()

        def body(j, carry):
            pltpu.async_copy(h2_hbm.at[src_v.at[j]], buf_v, sem).wait()
            pltpu.sync_copy(buf_v, acc.at[dst_v.at[j]], add=True)
            return carry

        lax.fori_loop(0, EPT_CHUNKS, body, 0)
        plsc.subcore_barrier()
        for k in range(RPT // C):
            off = sid * RPT + k * C
            pltpu.sync_copy(acc.at[pl.ds(off, C)], buf_v)
            pltpu.sync_copy(buf_v, acc_hbm.at[cid].at[pl.ds(off, C)])

    pl.run_scoped(scope, pltpu.VMEM_SHARED((N_PAD, D), jnp.float32))


def _mm_body(x_ref, w_ref, deg_ref, h2_ref):
    deg = deg_ref[0, :, 0] + deg_ref[1, :, 0] + 1.0
    dis = lax.rsqrt(deg)
    h = jnp.dot(x_ref[...], w_ref[...], preferred_element_type=jnp.float32)
    h2_ref[...] = h * dis[:, None]


def _final_body(acc_ref, h2_ref, deg_ref, b_ref, out_ref):
    deg = deg_ref[0, :, 0] + deg_ref[1, :, 0] + 1.0
    dis = lax.rsqrt(deg)
    s = acc_ref[0] + acc_ref[1] + h2_ref[...]
    out_ref[...] = s * dis[:, None] + b_ref[...]


_MM_BLK = 400


def _matmul_scale(x, W, deg2):
    return pl.pallas_call(
        _mm_body,
        grid=(N // _MM_BLK,),
        in_specs=[
            pl.BlockSpec((_MM_BLK, D), lambda i: (i, 0)),
            pl.BlockSpec((D, D), lambda i: (0, 0)),
            pl.BlockSpec((NC, _MM_BLK, DEG_W), lambda i: (0, i, 0)),
        ],
        out_specs=pl.BlockSpec((_MM_BLK, D), lambda i: (i, 0)),
        out_shape=jax.ShapeDtypeStruct((N, D), jnp.float32),
    )(x, W, deg2)


def _combine(acc2, h2, deg2, b2):
    return pl.pallas_call(
        _final_body,
        grid=(N // _MM_BLK,),
        in_specs=[
            pl.BlockSpec((NC, _MM_BLK, D), lambda i: (0, i, 0)),
            pl.BlockSpec((_MM_BLK, D), lambda i: (i, 0)),
            pl.BlockSpec((NC, _MM_BLK, DEG_W), lambda i: (0, i, 0)),
            pl.BlockSpec((1, D), lambda i: (0, 0)),
        ],
        out_specs=pl.BlockSpec((_MM_BLK, D), lambda i: (i, 0)),
        out_shape=jax.ShapeDtypeStruct((N, D), jnp.float32),
    )(acc2, h2, deg2, b2)


def kernel(x, edge_index, W, b):
    src = edge_index[0]
    dst = edge_index[1]
    pad = E_PAD - E
    srcr = jnp.concatenate(
        [src, jnp.zeros((pad,), jnp.int32)]).reshape(NW, EPT_CHUNKS, C)
    dstr = jnp.concatenate(
        [dst, jnp.full((pad,), N, jnp.int32)]).reshape(NW, EPT_CHUNKS, C)
    ones_deg = jnp.ones((C, DEG_W), jnp.float32)
    zeros_deg = jnp.zeros((RPT, DEG_W), jnp.float32)
    zeros_row = jnp.zeros((C, D), jnp.float32)

    deg2 = _deg_kernel(dstr, ones_deg, zeros_deg)
    h2 = _matmul_scale(x, W, deg2)
    acc2 = _scatter_kernel(h2, srcr, dstr, zeros_row)
    return _combine(acc2, h2, deg2, b.reshape(1, D))


# speedup vs baseline: 12.0381x; 12.0381x over previous
import functools

import jax
import jax.numpy as jnp
from jax import lax
from jax.experimental import pallas as pl
from jax.experimental.pallas import tpu as pltpu
from jax.experimental.pallas import tpu_sc as plsc

N = 10000
D = 128
E = 320000

NC, NS = 2, 16
NW = NC * NS
L = 16
C = 128
EPT_CHUNKS = 79
EPT = EPT_CHUNKS * C
E_PAD = NW * EPT
N_PAD = 10240
RPT = N_PAD // NS
HR = N_PAD // 128

_mesh = plsc.VectorSubcoreMesh(
    core_axis_name="c", subcore_axis_name="s", num_cores=NC, num_subcores=NS)


@functools.partial(
    pl.kernel,
    out_type=jax.ShapeDtypeStruct((NW, HR, 128), jnp.float32),
    mesh=_mesh,
    compiler_params=pltpu.CompilerParams(needs_layout_passes=False),
    scratch_types=[
        pltpu.VMEM((EPT_CHUNKS, C), jnp.int32),
        pltpu.VMEM((HR, 128), jnp.float32),
    ],
)
def _deg_kernel(dst_hbm, deg_hbm, dst_v, hist):
    cid = lax.axis_index("c")
    sid = lax.axis_index("s")
    tid = cid * NS + sid
    pltpu.sync_copy(dst_hbm.at[tid], dst_v)

    zero = jnp.zeros((L,), jnp.float32)

    def zbody(i, carry):
        for k in range(128 // L):
            hist[i, pl.ds(k * L, L)] = zero
        return carry

    lax.fori_loop(0, HR, zbody, 0)

    ones = jnp.ones((L,), jnp.float32)

    def body(j, carry):
        for k in range(C // L):
            d = dst_v[j, pl.ds(k * L, L)]
            plsc.addupdate_scatter(hist, [d >> 7, d & 127], ones)
        return carry

    lax.fori_loop(0, EPT_CHUNKS, body, 0)
    pltpu.sync_copy(hist, deg_hbm.at[tid])


@functools.partial(
    pl.kernel,
    out_type=jax.ShapeDtypeStruct((NC, N_PAD, D), jnp.float32),
    mesh=_mesh,
    scratch_types=[
        pltpu.VMEM((EPT_CHUNKS, C), jnp.int32),
        pltpu.VMEM((EPT_CHUNKS, C), jnp.int32),
        pltpu.VMEM((C, D), jnp.float32),
        pltpu.VMEM_SHARED((N_PAD, D), jnp.float32),
        pltpu.SemaphoreType.DMA,
    ],
)
def _scatter_kernel(h2_hbm, src_hbm, dst_hbm, zeros_hbm, acc_hbm,
                    src_v, dst_v, buf_v, acc, sem):
    cid = lax.axis_index("c")
    sid = lax.axis_index("s")
    tid = cid * NS + sid
    pltpu.sync_copy(src_hbm.at[tid], src_v)
    pltpu.sync_copy(dst_hbm.at[tid], dst_v)

    pltpu.sync_copy(zeros_hbm, buf_v)
    for k in range(RPT // C):
        pltpu.sync_copy(buf_v, acc.at[pl.ds(sid * RPT + k * C, C)])
    plsc.subcore_barrier()

    def body(j, carry):
        pltpu.async_copy(h2_hbm.at[src_v.at[j]], buf_v, sem).wait()
        pltpu.sync_copy(buf_v, acc.at[dst_v.at[j]], add=True)
        return carry

    lax.fori_loop(0, EPT_CHUNKS, body, 0)
    plsc.subcore_barrier()
    for k in range(RPT // C):
        off = sid * RPT + k * C
        pltpu.sync_copy(acc.at[pl.ds(off, C)], buf_v)
        pltpu.sync_copy(buf_v, acc_hbm.at[cid].at[pl.ds(off, C)])


_BLK = 640


def _mm_body(x_ref, w_ref, deg_ref, h2_ref):
    deg = jnp.sum(deg_ref[:, :, 0], axis=0) + 1.0
    dis = lax.rsqrt(deg)
    h = jnp.dot(x_ref[...], w_ref[...], preferred_element_type=jnp.float32)
    h2_ref[...] = h * dis[:, None]


def _final_body(acc_ref, h2_ref, deg_ref, b_ref, out_ref):
    deg = jnp.sum(deg_ref[:, :, 0], axis=0) + 1.0
    dis = lax.rsqrt(deg)
    s = acc_ref[0] + acc_ref[1] + h2_ref[...]
    out_ref[...] = s * dis[:, None] + b_ref[...]


def _matmul_scale(x_pad, W, degf):
    return pl.pallas_call(
        _mm_body,
        grid=(N_PAD // _BLK,),
        in_specs=[
            pl.BlockSpec((_BLK, D), lambda i: (i, 0)),
            pl.BlockSpec((D, D), lambda i: (0, 0)),
            pl.BlockSpec((NW, _BLK, 1), lambda i: (0, i, 0)),
        ],
        out_specs=pl.BlockSpec((_BLK, D), lambda i: (i, 0)),
        out_shape=jax.ShapeDtypeStruct((N_PAD, D), jnp.float32),
    )(x_pad, W, degf)


def _combine(acc2, h2, degf, b2):
    return pl.pallas_call(
        _final_body,
        grid=(N_PAD // _BLK,),
        in_specs=[
            pl.BlockSpec((NC, _BLK, D), lambda i: (0, i, 0)),
            pl.BlockSpec((_BLK, D), lambda i: (i, 0)),
            pl.BlockSpec((NW, _BLK, 1), lambda i: (0, i, 0)),
            pl.BlockSpec((1, D), lambda i: (0, 0)),
        ],
        out_specs=pl.BlockSpec((_BLK, D), lambda i: (i, 0)),
        out_shape=jax.ShapeDtypeStruct((N_PAD, D), jnp.float32),
    )(acc2, h2, degf, b2)


def kernel(x, edge_index, W, b):
    src = edge_index[0]
    dst = edge_index[1]
    pad = E_PAD - E
    srcr = jnp.concatenate(
        [src, jnp.zeros((pad,), jnp.int32)]).reshape(NW, EPT_CHUNKS, C)
    dstp = jnp.concatenate([dst, jnp.full((pad,), N, jnp.int32)])
    dstr = dstp.reshape(NW, EPT_CHUNKS, C)
    zeros_row = jnp.zeros((C, D), jnp.float32)
    x_pad = jnp.concatenate([x, jnp.zeros((N_PAD - N, D), jnp.float32)])

    deg = _deg_kernel(dstr)
    degf = deg.reshape(NW, N_PAD, 1)
    h2 = _matmul_scale(x_pad, W, degf)
    acc2 = _scatter_kernel(h2, srcr, dstr, zeros_row)
    out = _combine(acc2, h2, degf, b.reshape(1, D))
    return out[:N]
